# Initial kernel scaffold; baseline (speedup 1.0000x reference)
import jax
import jax.numpy as jnp
from jax.experimental import pallas as pl


def kernel(x, edge_index):
    raise NotImplementedError("write your pallas kernel here")


# speedup vs baseline: 6.8878x; 6.8878x over previous
import functools

import jax
import jax.numpy as jnp
from jax import lax
from jax.experimental import pallas as pl
from jax.experimental.pallas import tpu as pltpu
from jax.experimental.pallas import tpu_sc as plsc

N_NODES = 10000
N_EDGES = 320000
D_FEAT = 128

_NC = 2
_NS = 16
_NW = _NC * _NS

_EPW = N_EDGES // _NW
_B = 128
_NB = _EPW // _B
_TAIL = _EPW - _NB * _B
_RPT = 624
_RPT_EXTRA = N_NODES - _NS * _RPT


def _sc_scatter_gather(x_hbm, src_hbm, dst_hbm, part_hbm,
                       sidx, didx, rows, sidx_t, didx_t, rows_t, acc, sem):
    c = lax.axis_index("c")
    s = lax.axis_index("s")
    wid = s * _NC + c
    ebase = wid * _EPW

    zero16 = jnp.zeros((16,), jnp.float32)
    def zrow(r, carry):
        for k in range(D_FEAT // 16):
            rows[r, pl.ds(k * 16, 16)] = zero16
        return carry
    lax.fori_loop(0, _B, zrow, 0)
    z0 = s * _RPT
    nfull = _RPT // _B
    for k in range(nfull):
        pltpu.sync_copy(rows, acc.at[pl.ds(z0 + k * _B, _B)])
    rem = _RPT - nfull * _B
    if rem:
        pltpu.sync_copy(rows.at[pl.ds(0, rem)], acc.at[pl.ds(z0 + nfull * _B, rem)])

    @pl.when(s == _NS - 1)
    def _zero_extra():
        pltpu.sync_copy(rows.at[pl.ds(0, _RPT_EXTRA)],
                        acc.at[pl.ds(_NS * _RPT, _RPT_EXTRA)])
    plsc.subcore_barrier()

    def chunk(j, carry):
        off = ebase + j * _B
        pltpu.sync_copy(src_hbm.at[pl.ds(off, _B)], sidx)
        pltpu.sync_copy(dst_hbm.at[pl.ds(off, _B)], didx)
        pltpu.async_copy(x_hbm.at[sidx], rows, sem).wait()
        pltpu.sync_copy(rows, acc.at[didx], add=True)
        return carry
    lax.fori_loop(0, _NB, chunk, 0)

    if _TAIL:
        off = ebase + _NB * _B
        pltpu.sync_copy(src_hbm.at[pl.ds(off, _TAIL)], sidx_t)
        pltpu.sync_copy(dst_hbm.at[pl.ds(off, _TAIL)], didx_t)
        pltpu.async_copy(x_hbm.at[sidx_t], rows_t, sem).wait()
        pltpu.sync_copy(rows_t, acc.at[didx_t], add=True)

    plsc.subcore_barrier()

    pltpu.sync_copy(acc.at[pl.ds(z0, _RPT)], part_hbm.at[c, pl.ds(z0, _RPT)])

    @pl.when(s == _NS - 1)
    def _write_extra():
        pltpu.sync_copy(acc.at[pl.ds(_NS * _RPT, _RPT_EXTRA)],
                        part_hbm.at[c, pl.ds(_NS * _RPT, _RPT_EXTRA)])


def _combine_body(p_ref, o_ref):
    o_ref[...] = p_ref[0] + p_ref[1]


def kernel(x, edge_index):
    assert x.shape == (N_NODES, D_FEAT)
    src = edge_index[0].astype(jnp.int32)
    dst = edge_index[1].astype(jnp.int32)

    mesh = plsc.VectorSubcoreMesh(core_axis_name="c", subcore_axis_name="s")
    sc_call = pl.kernel(
        _sc_scatter_gather,
        out_type=jax.ShapeDtypeStruct((_NC, N_NODES, D_FEAT), jnp.float32),
        mesh=mesh,
        scratch_types=[
            pltpu.VMEM((_B,), jnp.int32),
            pltpu.VMEM((_B,), jnp.int32),
            pltpu.VMEM((_B, D_FEAT), jnp.float32),
            pltpu.VMEM((_TAIL,), jnp.int32),
            pltpu.VMEM((_TAIL,), jnp.int32),
            pltpu.VMEM((_TAIL, D_FEAT), jnp.float32),
            pltpu.VMEM_SHARED((N_NODES, D_FEAT), jnp.float32),
            pltpu.SemaphoreType.DMA,
        ],
    )
    partials = sc_call(x, src, dst)

    blk = 1000
    out = pl.pallas_call(
        _combine_body,
        out_shape=jax.ShapeDtypeStruct((N_NODES, D_FEAT), jnp.float32),
        grid=(N_NODES // blk,),
        in_specs=[pl.BlockSpec((_NC, blk, D_FEAT), lambda i: (0, i, 0))],
        out_specs=pl.BlockSpec((blk, D_FEAT), lambda i: (i, 0)),
    )(partials)
    return out
